# baseline (speedup 1.0000x reference)
import functools

import jax
import jax.numpy as jnp
from jax import lax
from jax.experimental import pallas as pl
from jax.experimental.pallas import tpu as pltpu
from jax.experimental.pallas import tpu_sc as plsc

_START_ID = 2
_PCNK = 16
_LANES = 16


def _build_emb_kernel(B, T, D, n_workers):
    pos_per_w = T // n_workers
    n_chunks = pos_per_w // _PCNK
    rows_per_chunk = B * _PCNK
    mesh = plsc.VectorSubcoreMesh(core_axis_name="c", subcore_axis_name="s")

    @functools.partial(
        pl.kernel,
        mesh=mesh,
        out_type=jax.ShapeDtypeStruct((B * T, D), jnp.float32),
        scratch_types=[
            pltpu.VMEM((B * 32,), jnp.int32),
            pltpu.VMEM((B * 32,), jnp.int32),
            pltpu.VMEM((rows_per_chunk,), jnp.int32),
            pltpu.VMEM((rows_per_chunk,), jnp.int32),
            pltpu.VMEM((rows_per_chunk, D), jnp.float32),
            pltpu.VMEM((rows_per_chunk, D), jnp.float32),
            pltpu.VMEM((_PCNK, D), jnp.float32),
            pltpu.VMEM((_PCNK, D), jnp.float32),
            pltpu.SemaphoreType.DMA,
            pltpu.SemaphoreType.DMA,
            pltpu.SemaphoreType.DMA,
            pltpu.SemaphoreType.DMA,
            pltpu.SemaphoreType.DMA,
            pltpu.SemaphoreType.DMA,
        ],
    )
    def emb(lab_hbm, wte_hbm, wpe_hbm, out_hbm,
            buf0, buf1, idx0, idx1, rows0, rows1, wrows0, wrows1,
            sl0, sl1, sg0, sg1, ss0, ss1):
        buf, idx, rows, wrows = [buf0, buf1], [idx0, idx1], [rows0, rows1], [wrows0, wrows1]
        sl, sg, ss = [sl0, sl1], [sg0, sg1], [ss0, ss1]
        wid = lax.axis_index("s") * 2 + lax.axis_index("c")
        pos_w0 = wid * pos_per_w
        lane = lax.iota(jnp.int32, _LANES)
        pend = {}

        def load(c):
            st = c % 2
            pos0 = pos_w0 + c * _PCNK
            s = (pos0 == 0).astype(jnp.int32)
            p = pend.setdefault(c, {})
            for b in range(B):
                p[f"lab{b}"] = pltpu.async_copy(
                    lab_hbm.at[pl.ds(b * T + pos0 - 8 + 8 * s, _PCNK + 8)],
                    buf[st].at[pl.ds(32 * b + 8 * s, _PCNK + 8)], sl[st])
            p["wpe"] = pltpu.async_copy(
                wpe_hbm.at[pl.ds(pos0, _PCNK)], wrows[st], sg[st])

        def launch_gather(c):
            st = c % 2
            pos0 = pos_w0 + c * _PCNK
            s = (pos0 == 0).astype(jnp.int32)
            for b in range(B):
                pend[c][f"lab{b}"].wait()
            keep = 1 - (1 - jnp.minimum(lane, 1)) * s
            for b in range(B):
                v = buf[st][pl.ds(32 * b + 7, _LANES)]
                v = v * keep + _START_ID * (1 - keep)
                idx[st][pl.ds(b * _LANES, _LANES)] = v
            pend[c]["gat"] = pltpu.async_copy(
                wte_hbm.at[idx[st]], rows[st], sg[st])

        def finish(c):
            st = c % 2
            pos0 = pos_w0 + c * _PCNK
            pend[c]["wpe"].wait()
            pend[c]["gat"].wait()

            def add_row(j, carry):
                r = lax.bitwise_and(j, _PCNK - 1)
                for k in range(D // _LANES):
                    sl_ = pl.ds(k * _LANES, _LANES)
                    rows[st][j, sl_] = rows[st][j, sl_] + wrows[st][r, sl_]
                return carry

            lax.fori_loop(0, rows_per_chunk, add_row, 0)
            p = pend[c]
            for b in range(B):
                p[f"st{b}"] = pltpu.async_copy(
                    rows[st].at[pl.ds(b * _LANES, _LANES)],
                    out_hbm.at[pl.ds(b * T + pos0, _PCNK)], ss[st])

        def drain_stores(c):
            for b in range(B):
                pend[c][f"st{b}"].wait()

        load(0)
        launch_gather(0)
        load(1)
        launch_gather(1)
        for c in range(n_chunks):
            finish(c)
            if c + 2 < n_chunks:
                drain_stores(c)
                load(c + 2)
                launch_gather(c + 2)
        drain_stores(n_chunks - 2)
        drain_stores(n_chunks - 1)

    return emb


def kernel(hidden, labels, wte_table, wpe_table):
    B, T = labels.shape
    D = wte_table.shape[1]
    info = plsc.get_sparse_core_info()
    n_workers = info.num_cores * info.num_subcores
    emb = _build_emb_kernel(B, T, D, n_workers)
    out_flat = emb(labels.reshape(B * T), wte_table, wpe_table)
    return (hidden, out_flat.reshape(B, T, D), labels)


# speedup vs baseline: 1.4341x; 1.4341x over previous
import functools

import jax
import jax.numpy as jnp
from jax import lax
from jax.experimental import pallas as pl
from jax.experimental.pallas import tpu as pltpu
from jax.experimental.pallas import tpu_sc as plsc

_START_ID = 2
_PCNK = 16
_RING = 4
_LANES = 16
_LWIN = 80


def _build_emb_kernel(B, T, D, n_workers):
    pos_per_w = T // n_workers
    n_pchunks = pos_per_w // _PCNK
    n_units = n_pchunks * B
    mesh = plsc.VectorSubcoreMesh(core_axis_name="c", subcore_axis_name="s")

    scratch = (
        [pltpu.VMEM((B * _LWIN,), jnp.int32)]
        + [pltpu.VMEM((_PCNK,), jnp.int32) for _ in range(_RING)]
        + [pltpu.VMEM((_PCNK, D), jnp.float32) for _ in range(_RING)]
        + [pltpu.VMEM((_PCNK, D), jnp.float32) for _ in range(2)]
        + [pltpu.SemaphoreType.DMA]
        + [pltpu.SemaphoreType.DMA for _ in range(_RING)]
        + [pltpu.SemaphoreType.DMA for _ in range(_RING)]
    )

    @functools.partial(
        pl.kernel,
        mesh=mesh,
        out_type=jax.ShapeDtypeStruct((B * T, D), jnp.float32),
        scratch_types=scratch,
    )
    def emb(lab_hbm, wte_hbm, wpe_hbm, out_hbm, buf, *rest):
        idx = rest[0:_RING]
        rows = rest[_RING:2 * _RING]
        wrows = rest[2 * _RING:2 * _RING + 2]
        slab = rest[2 * _RING + 2]
        sg = rest[2 * _RING + 3:2 * _RING + 3 + _RING]
        ss = rest[2 * _RING + 3 + _RING:]
        wid = lax.axis_index("s") * 2 + lax.axis_index("c")
        pos_w0 = wid * pos_per_w
        sw = (pos_w0 == 0).astype(jnp.int32)
        lane = lax.iota(jnp.int32, _LANES)
        pend = {}

        lab_cps = [
            pltpu.async_copy(
                lab_hbm.at[pl.ds(b * T + pos_w0 - 8 + 8 * sw, _LWIN - 8)],
                buf.at[pl.ds(_LWIN * b + 8 * sw, _LWIN - 8)], slab)
            for b in range(B)
        ]
        for cp in lab_cps:
            cp.wait()

        def launch(t):
            st = t % _RING
            c, b = divmod(t, B)
            pos0 = pos_w0 + c * _PCNK
            p = pend.setdefault(t, {})
            v = buf[pl.ds(_LWIN * b + 7 + _PCNK * c, _LANES)]
            if c == 0:
                keep = 1 - (1 - jnp.minimum(lane, 1)) * sw
                v = v * keep + _START_ID * (1 - keep)
            idx[st][pl.ds(0, _LANES)] = v
            p["gat"] = pltpu.async_copy(wte_hbm.at[idx[st]], rows[st], sg[st])
            if b == 0:
                p["wpe"] = pltpu.async_copy(
                    wpe_hbm.at[pl.ds(pos0, _PCNK)], wrows[c % 2], sg[st])

        def finish(t):
            st = t % _RING
            c, b = divmod(t, B)
            pos0 = pos_w0 + c * _PCNK
            pend[t]["gat"].wait()
            if b == 0:
                pend[t]["wpe"].wait()
            wr = wrows[c % 2]

            def add_row(j, carry):
                for k in range(D // _LANES):
                    sl = pl.ds(k * _LANES, _LANES)
                    rows[st][j, sl] = rows[st][j, sl] + wr[j, sl]
                return carry

            lax.fori_loop(0, _PCNK, add_row, 0)
            pend[t]["st"] = pltpu.async_copy(
                rows[st], out_hbm.at[pl.ds(b * T + pos0, _PCNK)], ss[st])

        for t in range(_RING):
            launch(t)
        for t in range(n_units):
            finish(t)
            if t + _RING < n_units:
                pend[t]["st"].wait()
                launch(t + _RING)
        for t in range(n_units - _RING, n_units):
            pend[t]["st"].wait()

    return emb


def kernel(hidden, labels, wte_table, wpe_table):
    B, T = labels.shape
    D = wte_table.shape[1]
    info = plsc.get_sparse_core_info()
    n_workers = info.num_cores * info.num_subcores
    emb = _build_emb_kernel(B, T, D, n_workers)
    out_flat = emb(labels.reshape(B * T), wte_table, wpe_table)
    return (hidden, out_flat.reshape(B, T, D), labels)


# speedup vs baseline: 1.5128x; 1.0549x over previous
import functools

import jax
import jax.numpy as jnp
from jax import lax
from jax.experimental import pallas as pl
from jax.experimental.pallas import tpu as pltpu
from jax.experimental.pallas import tpu_sc as plsc

_START_ID = 2
_PCNK = 16
_RING = 6
_LANES = 16
_LWIN = 80


def _build_emb_kernel(B, T, D, n_workers):
    pos_per_w = T // n_workers
    n_pchunks = pos_per_w // _PCNK
    n_units = n_pchunks * B
    mesh = plsc.VectorSubcoreMesh(core_axis_name="c", subcore_axis_name="s")

    scratch = (
        [pltpu.VMEM((B * _LWIN,), jnp.int32)]
        + [pltpu.VMEM((_PCNK,), jnp.int32) for _ in range(_RING)]
        + [pltpu.VMEM((_PCNK, D), jnp.float32) for _ in range(_RING)]
        + [pltpu.VMEM((_PCNK, D), jnp.float32) for _ in range(3)]
        + [pltpu.SemaphoreType.DMA]
        + [pltpu.SemaphoreType.DMA for _ in range(_RING)]
        + [pltpu.SemaphoreType.DMA for _ in range(_RING)]
    )

    @functools.partial(
        pl.kernel,
        mesh=mesh,
        out_type=jax.ShapeDtypeStruct((B * T, D), jnp.float32),
        scratch_types=scratch,
    )
    def emb(lab_hbm, wte_hbm, wpe_hbm, out_hbm, buf, *rest):
        idx = rest[0:_RING]
        rows = rest[_RING:2 * _RING]
        wrows = rest[2 * _RING:2 * _RING + 3]
        slab = rest[2 * _RING + 3]
        sg = rest[2 * _RING + 4:2 * _RING + 4 + _RING]
        ss = rest[2 * _RING + 4 + _RING:]
        wid = lax.axis_index("s") * 2 + lax.axis_index("c")
        pos_w0 = wid * pos_per_w
        sw = (pos_w0 == 0).astype(jnp.int32)
        lane = lax.iota(jnp.int32, _LANES)
        pend = {}

        lab_cps = [
            pltpu.async_copy(
                lab_hbm.at[pl.ds(b * T + pos_w0 - 8 + 8 * sw, _LWIN - 8)],
                buf.at[pl.ds(_LWIN * b + 8 * sw, _LWIN - 8)], slab)
            for b in range(B)
        ]
        for cp in lab_cps:
            cp.wait()

        def launch(t):
            st = t % _RING
            c, b = divmod(t, B)
            pos0 = pos_w0 + c * _PCNK
            p = pend.setdefault(t, {})
            v = buf[pl.ds(_LWIN * b + 7 + _PCNK * c, _LANES)]
            if c == 0:
                keep = 1 - (1 - jnp.minimum(lane, 1)) * sw
                v = v * keep + _START_ID * (1 - keep)
            idx[st][pl.ds(0, _LANES)] = v
            p["gat"] = pltpu.async_copy(wte_hbm.at[idx[st]], rows[st], sg[st])
            if b == 0:
                p["wpe"] = pltpu.async_copy(
                    wpe_hbm.at[pl.ds(pos0, _PCNK)], wrows[c % 3], sg[st])

        def finish(t):
            st = t % _RING
            c, b = divmod(t, B)
            pos0 = pos_w0 + c * _PCNK
            pend[t]["gat"].wait()
            if b == 0:
                pend[t]["wpe"].wait()
            wr = wrows[c % 3]

            def add_row(j, carry):
                for k in range(D // _LANES):
                    sl = pl.ds(k * _LANES, _LANES)
                    rows[st][j, sl] = rows[st][j, sl] + wr[j, sl]
                return carry

            lax.fori_loop(0, _PCNK, add_row, 0)
            pend[t]["st"] = pltpu.async_copy(
                rows[st], out_hbm.at[pl.ds(b * T + pos0, _PCNK)], ss[st])

        for t in range(_RING):
            launch(t)
        for t in range(n_units):
            finish(t)
            if t + _RING < n_units:
                pend[t]["st"].wait()
                launch(t + _RING)
        for t in range(n_units - _RING, n_units):
            pend[t]["st"].wait()

    return emb


def kernel(hidden, labels, wte_table, wpe_table):
    B, T = labels.shape
    D = wte_table.shape[1]
    info = plsc.get_sparse_core_info()
    n_workers = info.num_cores * info.num_subcores
    emb = _build_emb_kernel(B, T, D, n_workers)
    out_flat = emb(labels.reshape(B * T), wte_table, wpe_table)
    return (hidden, out_flat.reshape(B, T, D), labels)
